# baseline (speedup 1.0000x reference)
import functools

import numpy as np
import jax
import jax.numpy as jnp
from jax import lax
from jax.experimental import pallas as pl

_BN_DIV = np.float32(np.sqrt(np.float32(1.0 + 1e-5)))
_HIGH = lax.Precision.HIGHEST

_LAYERS = [
    (1, 32, 1024, 80, 512, 40),
    (32, 32, 512, 40, 256, 20),
    (32, 64, 256, 20, 128, 10),
    (64, 64, 128, 10, 64, 5),
    (64, 128, 64, 5, 32, 3),
    (128, 128, 32, 3, 16, 2),
]

_TAP = [(0, 0), (1, 0), (0, 1)]


def _pad_planes(x, Ho, Wo):
    B, H, W, C = x.shape
    U, V = Ho + 1, Wo + 1
    xp = jnp.pad(x, ((0, 0), (1, 2 * Ho + 1 - H), (1, 2 * Wo + 1 - W), (0, 0)))
    xr = xp.reshape(B, U, 2, V, 2, C)
    return [xr[:, :, pi, :, pj, :].reshape(B, U * V, C)
            for pi in (0, 1) for pj in (0, 1)]


def _conv_body(p4, w4, g, b, out, *, M, V, MT, gridded):
    shifts = (0, 1, V, V + 1)
    gv = g[...]
    bias = b[...]
    for m0 in range(0, M, MT):
        mt = min(MT, M - m0)
        acc = None
        for si, s in enumerate(shifts):
            if gridded:
                tap = p4[0, s + m0: s + m0 + mt, :]
            else:
                tap = p4[s + m0: s + m0 + mt, :]
            c = jnp.dot(tap, w4[si], preferred_element_type=jnp.float32)
            acc = c if acc is None else acc + c
        y = jnp.maximum(gv * acc / _BN_DIV + bias, 0.0)
        if gridded:
            out[0, m0:m0 + mt, :] = y
        else:
            out[m0:m0 + mt, :] = y


def _conv0_body(p, w, g, b, out):
    acc = jnp.dot(p[0], w[...], preferred_element_type=jnp.float32)
    y = g[...] * acc / _BN_DIV + b[...]
    out[0] = jnp.maximum(y, 0.0)


def _conv_layer(x, w, g, b, idx):
    Cin, Cout, H, W, Ho, Wo = _LAYERS[idx]
    B = x.shape[0]
    U, V = Ho + 1, Wo + 1
    wt = w.transpose(2, 3, 1, 0)
    g2 = g.reshape(1, Cout)
    b2 = b.reshape(1, Cout)

    if idx == 0:
        xp = jnp.pad(x[..., 0], ((0, 0), (1, 2 * Ho + 1 - H), (1, 2 * Wo + 1 - W)))
        taps = [xp[:, kh:kh + 2 * Ho:2, kw:kw + 2 * Wo:2]
                for kh in range(3) for kw in range(3)]
        patches = jnp.stack(taps, axis=-1).reshape(B, Ho * Wo, 9)
        w9 = wt.reshape(9, Cout)
        MT = 2560
        out = pl.pallas_call(
            _conv0_body,
            grid=(B, (Ho * Wo) // MT),
            in_specs=[
                pl.BlockSpec((1, MT, 9), lambda i, j: (i, j, 0)),
                pl.BlockSpec((9, Cout), lambda i, j: (0, 0)),
                pl.BlockSpec((1, Cout), lambda i, j: (0, 0)),
                pl.BlockSpec((1, Cout), lambda i, j: (0, 0)),
            ],
            out_specs=pl.BlockSpec((1, MT, Cout), lambda i, j: (i, j, 0)),
            out_shape=jax.ShapeDtypeStruct((B, Ho * Wo, Cout), jnp.float32),
        )(patches, w9, g2, b2)
        return out.reshape(B, Ho, Wo, Cout)

    planes = _pad_planes(x, Ho, Wo)
    p4 = jnp.concatenate(planes, axis=-1)
    kmap = {(0, 0): 0, (0, 1): 1, (1, 0): 2}
    blocks = []
    for du, dv in ((0, 0), (0, 1), (1, 0), (1, 1)):
        rows = []
        for pi in (0, 1):
            for pj in (0, 1):
                if (du, pi) in kmap and (dv, pj) in kmap:
                    rows.append(wt[kmap[(du, pi)], kmap[(dv, pj)]])
                else:
                    rows.append(jnp.zeros((Cin, Cout), jnp.float32))
        blocks.append(jnp.concatenate(rows, axis=0))
    w4 = jnp.stack(blocks, axis=0)

    gridded = idx <= 2
    MT = 256
    if gridded:
        Mg = Ho * V
        body = functools.partial(_conv_body, M=Mg, V=V, MT=MT, gridded=True)
        p4 = jnp.pad(p4, ((0, 0), (0, 8), (0, 0)))
        UVp = U * V + 8
        out = pl.pallas_call(
            body,
            grid=(B,),
            in_specs=[
                pl.BlockSpec((1, UVp, 4 * Cin), lambda i: (i, 0, 0)),
                pl.BlockSpec((4, 4 * Cin, Cout), lambda i: (0, 0, 0)),
                pl.BlockSpec((1, Cout), lambda i: (0, 0)),
                pl.BlockSpec((1, Cout), lambda i: (0, 0)),
            ],
            out_specs=pl.BlockSpec((1, Mg, Cout), lambda i: (i, 0, 0)),
            out_shape=jax.ShapeDtypeStruct((B, Mg, Cout), jnp.float32),
        )(p4, w4, g2, b2)
        out = out.reshape(B, Ho, V, Cout)[:, :, :Wo, :]
    else:
        M = B * U * V
        body = functools.partial(_conv_body, M=M, V=V, MT=MT, gridded=False)
        p4 = jnp.pad(p4.reshape(M, 4 * Cin), ((0, 8), (0, 0)))
        out = pl.pallas_call(
            body,
            out_shape=jax.ShapeDtypeStruct((M, Cout), jnp.float32),
        )(p4, w4, g2, b2)
        out = out.reshape(B, U, V, Cout)[:, :Ho, :Wo, :]
    return out


def _gru_vq_body(hs, wih, whh, bih, bhh, cb1, cb2, cb3,
                 zq1o, zq2o, zq3o, zsumo, i1o, losso):
    T, B, D = 16, 32, 256
    hs_v = hs[...]
    gi_all = lax.dot_general(hs_v, wih[...], (((1,), (1,)), ((), ())),
                             preferred_element_type=jnp.float32) + bih[...]
    h = jnp.zeros((B, D), jnp.float32)
    for t in range(T):
        gi = gi_all[t * B:(t + 1) * B, :]
        gh = lax.dot_general(h, whh[...], (((1,), (1,)), ((), ())),
                             preferred_element_type=jnp.float32) + bhh[...]
        r = jax.nn.sigmoid(gi[:, 0:D] + gh[:, 0:D])
        z = jax.nn.sigmoid(gi[:, D:2 * D] + gh[:, D:2 * D])
        n = jnp.tanh(gi[:, 2 * D:3 * D] + r * gh[:, 2 * D:3 * D])
        h = (1.0 - z) * n + z * h

    res = h
    loss = jnp.zeros((), jnp.float32)
    outs = []
    K = 1024
    ones = jnp.ones((1, D), jnp.float32)
    ii = lax.broadcasted_iota(jnp.int32, (B, K), 1)
    for k, cb in enumerate((cb1, cb2, cb3)):
        emb = cb[...]
        e2 = lax.dot_general(ones, emb * emb, (((1,), (1,)), ((), ())),
                             precision=_HIGH,
                             preferred_element_type=jnp.float32)
        z2 = jnp.sum(res * res, axis=1, keepdims=True)
        s = lax.dot_general(res, emb, (((1,), (1,)), ((), ())),
                            preferred_element_type=jnp.float32)
        d = z2 + e2 - 2.0 * s
        m = jnp.min(d, axis=1, keepdims=True)
        cand = jnp.where(d == m, ii, K)
        idxc = jnp.min(cand, axis=1, keepdims=True)
        if k == 0:
            i1o[...] = idxc
        enc = (ii == idxc).astype(jnp.float32)
        zq = lax.dot_general(enc, emb, (((1,), (0,)), ((), ())),
                             preferred_element_type=jnp.float32)
        diff = zq - res
        loss = loss + 1.25 * jnp.mean(diff * diff)
        zq = res + (zq - res)
        outs.append(zq)
        res = res - zq
    zq1o[...] = outs[0]
    zq2o[...] = outs[1]
    zq3o[...] = outs[2]
    zsumo[...] = outs[0] + outs[1] + outs[2]
    losso[...] = jnp.zeros((1, 1), jnp.float32) + loss


def kernel(speech, conv_w0, conv_w1, conv_w2, conv_w3, conv_w4, conv_w5,
           bn_g0, bn_g1, bn_g2, bn_g3, bn_g4, bn_g5,
           bn_b0, bn_b1, bn_b2, bn_b3, bn_b4, bn_b5,
           w_ih, w_hh, b_ih, b_hh, cb1, cb2, cb3):
    x = speech[..., None]
    conv_ws = [conv_w0, conv_w1, conv_w2, conv_w3, conv_w4, conv_w5]
    bn_gs = [bn_g0, bn_g1, bn_g2, bn_g3, bn_g4, bn_g5]
    bn_bs = [bn_b0, bn_b1, bn_b2, bn_b3, bn_b4, bn_b5]
    for i in range(6):
        x = _conv_layer(x, conv_ws[i], bn_gs[i], bn_bs[i], i)
    B, T, F, C = x.shape
    hs = x.transpose(1, 0, 3, 2).reshape(T * B, C * F)

    zq1, zq2, zq3, zsum, i1r, lossm = pl.pallas_call(
        _gru_vq_body,
        out_shape=(
            jax.ShapeDtypeStruct((32, 256), jnp.float32),
            jax.ShapeDtypeStruct((32, 256), jnp.float32),
            jax.ShapeDtypeStruct((32, 256), jnp.float32),
            jax.ShapeDtypeStruct((32, 256), jnp.float32),
            jax.ShapeDtypeStruct((32, 1), jnp.int32),
            jax.ShapeDtypeStruct((1, 1), jnp.float32),
        ),
    )(hs, w_ih, w_hh, b_ih.reshape(1, 768), b_hh.reshape(1, 768),
      cb1, cb2, cb3)

    z_q_out = jnp.concatenate([zq1, zq2, zq3], axis=1)
    vq_loss = lossm[0, 0]
    i1 = i1r
    codebooks = (zq1, zq2, zq3, zsum)
    return z_q_out, vq_loss, i1, codebooks
